# baseline (speedup 1.0000x reference)
import functools

import jax
import jax.numpy as jnp
from jax import lax
from jax.experimental import pallas as pl
from jax.experimental.pallas import tpu as pltpu
from jax.experimental.pallas import tpu_sc as plsc

D_ = 256
B_ = 16384

NC_ = 2
NS_ = 16
NW_ = NC_ * NS_
TOT_ = 2 * B_
ROWS_PER_W_ = TOT_ // NW_
CH_ = 128
N_CHUNKS_ = ROWS_PER_W_ // CH_


def _sc_gather(cat_idx, table):
    mesh = plsc.VectorSubcoreMesh(
        core_axis_name="c", subcore_axis_name="s",
        num_cores=NC_, num_subcores=NS_)

    @functools.partial(
        pl.kernel,
        out_type=jax.ShapeDtypeStruct((TOT_, D_), jnp.float32),
        mesh=mesh,
        scratch_types=[
            pltpu.VMEM((CH_,), jnp.int32),
            pltpu.VMEM((CH_,), jnp.int32),
            pltpu.VMEM((CH_, D_), jnp.float32),
            pltpu.VMEM((CH_, D_), jnp.float32),
            pltpu.SemaphoreType.DMA,
            pltpu.SemaphoreType.DMA,
        ],
    )
    def gk(idx_hbm, tab_hbm, out_hbm, idx_a, idx_b, rows_a, rows_b, sem_a, sem_b):
        wid = lax.axis_index("s") * NC_ + lax.axis_index("c")
        base = wid * ROWS_PER_W_
        idx_v = (idx_a, idx_b)
        rows_v = (rows_a, rows_b)
        sems = (sem_a, sem_b)
        pltpu.sync_copy(idx_hbm.at[pl.ds(base, CH_)], idx_a)
        gather = pltpu.async_copy(tab_hbm.at[idx_a], rows_a, sem_a)
        for c in range(N_CHUNKS_):
            cur = c % 2
            nxt = (c + 1) % 2
            if c + 1 < N_CHUNKS_:
                off_n = base + (c + 1) * CH_
                pltpu.sync_copy(idx_hbm.at[pl.ds(off_n, CH_)], idx_v[nxt])
                next_gather = pltpu.async_copy(
                    tab_hbm.at[idx_v[nxt]], rows_v[nxt], sems[nxt])
            gather.wait()
            off = base + c * CH_
            pltpu.sync_copy(rows_v[cur], out_hbm.at[pl.ds(off, CH_)])
            if c + 1 < N_CHUNKS_:
                gather = next_gather

    return gk(cat_idx, table)


BB_ = 512


def _mlp_body(ei_ref, ej_ref, w1_ref, b1_ref, w2_ref, b2_ref, out_ref):
    d = ei_ref[...] - ej_ref[...]
    x = d * d
    h = jnp.dot(x, w1_ref[...], preferred_element_type=jnp.float32)
    h = jnp.maximum(h + b1_ref[...], 0.0)
    out_ref[...] = (
        jnp.dot(h, w2_ref[...], preferred_element_type=jnp.float32)
        + b2_ref[0, 0])


def _tc_mlp(gathered, W1, b1, W2, b2):
    nb = B_ // BB_
    return pl.pallas_call(
        _mlp_body,
        grid=(nb,),
        in_specs=[
            pl.BlockSpec((BB_, D_), lambda i: (i, 0)),
            pl.BlockSpec((BB_, D_), lambda i: (i + nb, 0)),
            pl.BlockSpec((D_, D_), lambda i: (0, 0)),
            pl.BlockSpec((1, D_), lambda i: (0, 0)),
            pl.BlockSpec((D_, 1), lambda i: (0, 0)),
            pl.BlockSpec(memory_space=pltpu.SMEM),
        ],
        out_specs=pl.BlockSpec((BB_, 1), lambda i: (i, 0)),
        out_shape=jax.ShapeDtypeStruct((B_, 1), jnp.float32),
    )(gathered, gathered, W1, b1.reshape(1, D_), W2, b2.reshape(1, 1))


def kernel(p_vertices, q_vertices, embds, W1, b1, W2, b2):
    cat_idx = jnp.concatenate([p_vertices, q_vertices], axis=0)
    gathered = _sc_gather(cat_idx, embds)
    pred = _tc_mlp(gathered, W1, b1, W2, b2)
    return pred[:, 0]


# speedup vs baseline: 1.0668x; 1.0668x over previous
import functools

import jax
import jax.numpy as jnp
from jax import lax
from jax.experimental import pallas as pl
from jax.experimental.pallas import tpu as pltpu
from jax.experimental.pallas import tpu_sc as plsc

D_ = 256
B_ = 16384
L_ = 16

NC_ = 2
NS_ = 16
NW_ = NC_ * NS_
PAIRS_PER_W_ = B_ // NW_
CH_ = 64
N_CHUNKS_ = PAIRS_PER_W_ // CH_


def _sc_gather_sqdiff(p_idx, q_idx, table):
    mesh = plsc.VectorSubcoreMesh(
        core_axis_name="c", subcore_axis_name="s",
        num_cores=NC_, num_subcores=NS_)

    @functools.partial(
        pl.kernel,
        out_type=jax.ShapeDtypeStruct((B_, D_), jnp.float32),
        mesh=mesh,
        scratch_types=[
            pltpu.VMEM((PAIRS_PER_W_,), jnp.int32),
            pltpu.VMEM((PAIRS_PER_W_,), jnp.int32),
            pltpu.VMEM((CH_, D_), jnp.float32),
            pltpu.VMEM((CH_, D_), jnp.float32),
            pltpu.VMEM((CH_, D_), jnp.float32),
            pltpu.VMEM((CH_, D_), jnp.float32),
            pltpu.SemaphoreType.DMA,
            pltpu.SemaphoreType.DMA,
            pltpu.SemaphoreType.DMA,
            pltpu.SemaphoreType.DMA,
        ],
    )
    def gk(pidx_hbm, qidx_hbm, tab_hbm, out_hbm,
           pidx_v, qidx_v, bp0, bp1, bq0, bq1, gs0, gs1, ss0, ss1):
        wid = lax.axis_index("s") * NC_ + lax.axis_index("c")
        base = wid * PAIRS_PER_W_
        bp = (bp0, bp1)
        bq = (bq0, bq1)
        gsem = (gs0, gs1)
        ssem = (ss0, ss1)

        pltpu.sync_copy(pidx_hbm.at[pl.ds(base, PAIRS_PER_W_)], pidx_v)
        pltpu.sync_copy(qidx_hbm.at[pl.ds(base, PAIRS_PER_W_)], qidx_v)

        def fire_gather(c, s):
            isl = pl.ds(c * CH_, CH_)
            hp = pltpu.async_copy(tab_hbm.at[pidx_v.at[isl]], bp[s], gsem[s])
            hq = pltpu.async_copy(tab_hbm.at[qidx_v.at[isl]], bq[s], gsem[s])
            return (hp, hq)

        def compute(s):
            bps, bqs = bp[s], bq[s]

            @plsc.parallel_loop(0, CH_)
            def _(r):
                for k in range(D_ // L_):
                    sl = pl.ds(k * L_, L_)
                    d = bps[r, sl] - bqs[r, sl]
                    bps[r, sl] = d * d

        pend = [None, None]
        drain = [None, None]
        pend[0] = fire_gather(0, 0)
        for c in range(N_CHUNKS_):
            s = c % 2
            o = 1 - s
            for h in pend[s]:
                h.wait()
            if c + 1 < N_CHUNKS_:
                if drain[o] is not None:
                    drain[o].wait()
                pend[o] = fire_gather(c + 1, o)
            compute(s)
            drain[s] = pltpu.async_copy(
                bp[s], out_hbm.at[pl.ds(base + c * CH_, CH_)], ssem[s])
        drain[0].wait()
        drain[1].wait()

    return gk(p_idx, q_idx, table)


BB_ = 512


def _mlp_body(x_ref, w1_ref, b1_ref, w2r_ref, b2_ref, out_ref):
    xb = x_ref[...].astype(jnp.bfloat16)
    h = jnp.dot(xb, w1_ref[...], preferred_element_type=jnp.float32)
    h = jnp.maximum(h + b1_ref[...], 0.0)
    out_ref[...] = (
        jnp.sum(h * w2r_ref[...], axis=1, keepdims=True) + b2_ref[0, 0])


def _tc_mlp(x, W1bf, b1, W2r, b2):
    nb = B_ // BB_
    return pl.pallas_call(
        _mlp_body,
        grid=(nb,),
        in_specs=[
            pl.BlockSpec((BB_, D_), lambda i: (i, 0)),
            pl.BlockSpec((D_, D_), lambda i: (0, 0)),
            pl.BlockSpec((1, D_), lambda i: (0, 0)),
            pl.BlockSpec((1, D_), lambda i: (0, 0)),
            pl.BlockSpec(memory_space=pltpu.SMEM),
        ],
        out_specs=pl.BlockSpec((BB_, 1), lambda i: (i, 0)),
        out_shape=jax.ShapeDtypeStruct((B_, 1), jnp.float32),
    )(x, W1bf, b1.reshape(1, D_), W2r, b2.reshape(1, 1))


def kernel(p_vertices, q_vertices, embds, W1, b1, W2, b2):
    x = _sc_gather_sqdiff(p_vertices, q_vertices, embds)
    pred = _tc_mlp(x, W1.astype(jnp.bfloat16), b1,
                   W2.reshape(1, D_), b2)
    return pred[:, 0]


# speedup vs baseline: 1.3156x; 1.2332x over previous
import functools

import jax
import jax.numpy as jnp
from jax import lax
from jax.experimental import pallas as pl
from jax.experimental.pallas import tpu as pltpu
from jax.experimental.pallas import tpu_sc as plsc

D_ = 256
B_ = 16384
L_ = 16

NC_ = 2
NS_ = 16
NW_ = NC_ * NS_
PAIRS_PER_W_ = B_ // NW_
CH_ = 64
N_CHUNKS_ = PAIRS_PER_W_ // CH_


def _sc_gather_sqdiff(p_idx, q_idx, table):
    mesh = plsc.VectorSubcoreMesh(
        core_axis_name="c", subcore_axis_name="s",
        num_cores=NC_, num_subcores=NS_)

    @functools.partial(
        pl.kernel,
        out_type=jax.ShapeDtypeStruct((B_, D_), jnp.float32),
        mesh=mesh,
        scratch_types=[
            pltpu.VMEM((PAIRS_PER_W_,), jnp.int32),
            pltpu.VMEM((PAIRS_PER_W_,), jnp.int32),
            pltpu.VMEM((CH_, D_), jnp.float32),
            pltpu.VMEM((CH_, D_), jnp.float32),
            pltpu.VMEM((CH_, D_), jnp.float32),
            pltpu.VMEM((CH_, D_), jnp.float32),
            pltpu.SemaphoreType.DMA,
            pltpu.SemaphoreType.DMA,
            pltpu.SemaphoreType.DMA,
            pltpu.SemaphoreType.DMA,
        ],
    )
    def gk(pidx_hbm, qidx_hbm, tab_hbm, out_hbm,
           pidx_v, qidx_v, bp0, bp1, bq0, bq1, gs0, gs1, ss0, ss1):
        wid = lax.axis_index("s") * NC_ + lax.axis_index("c")
        base = wid * PAIRS_PER_W_
        bp = (bp0, bp1)
        bq = (bq0, bq1)
        gsem = (gs0, gs1)
        ssem = (ss0, ss1)

        pltpu.sync_copy(pidx_hbm.at[pl.ds(base, PAIRS_PER_W_)], pidx_v)
        pltpu.sync_copy(qidx_hbm.at[pl.ds(base, PAIRS_PER_W_)], qidx_v)

        def fire_gather(c, s):
            isl = pl.ds(c * CH_, CH_)
            hp = pltpu.async_copy(tab_hbm.at[pidx_v.at[isl]], bp[s], gsem[s])
            hq = pltpu.async_copy(tab_hbm.at[qidx_v.at[isl]], bq[s], gsem[s])
            return (hp, hq)

        def compute(s):
            bps, bqs = bp[s], bq[s]

            @plsc.parallel_loop(0, CH_)
            def _(r):
                for k in range(D_ // L_):
                    sl = pl.ds(k * L_, L_)
                    d = bps[r, sl] - bqs[r, sl]
                    bps[r, sl] = d * d

        pend = [None, None]
        drain = [None, None]
        pend[0] = fire_gather(0, 0)
        for c in range(N_CHUNKS_):
            s = c % 2
            o = 1 - s
            for h in pend[s]:
                h.wait()
            if c + 1 < N_CHUNKS_:
                if drain[o] is not None:
                    drain[o].wait()
                pend[o] = fire_gather(c + 1, o)
            compute(s)
            drain[s] = pltpu.async_copy(
                bp[s], out_hbm.at[pl.ds(base + c * CH_, CH_)], ssem[s])
        drain[0].wait()
        drain[1].wait()

    return gk(p_idx, q_idx, table)


BB_ = 2048


def _mlp_body(x_ref, w1_ref, b1_ref, w2r_ref, b2_ref, out_ref):
    xb = x_ref[...].astype(jnp.bfloat16)
    h = jnp.dot(xb, w1_ref[...], preferred_element_type=jnp.float32)
    h = jnp.maximum(h + b1_ref[...], 0.0)
    out_ref[...] = (
        jnp.sum(h * w2r_ref[...], axis=1, keepdims=True) + b2_ref[0, 0])


def _tc_mlp(x, W1bf, b1, W2r, b2):
    nb = B_ // BB_
    return pl.pallas_call(
        _mlp_body,
        grid=(nb,),
        in_specs=[
            pl.BlockSpec((BB_, D_), lambda i: (i, 0)),
            pl.BlockSpec((D_, D_), lambda i: (0, 0)),
            pl.BlockSpec((1, D_), lambda i: (0, 0)),
            pl.BlockSpec((1, D_), lambda i: (0, 0)),
            pl.BlockSpec(memory_space=pltpu.SMEM),
        ],
        out_specs=pl.BlockSpec((BB_, 1), lambda i: (i, 0)),
        out_shape=jax.ShapeDtypeStruct((B_, 1), jnp.float32),
    )(x, W1bf, b1.reshape(1, D_), W2r, b2.reshape(1, 1))


def kernel(p_vertices, q_vertices, embds, W1, b1, W2, b2):
    x = _sc_gather_sqdiff(p_vertices, q_vertices, embds)
    pred = _tc_mlp(x, W1.astype(jnp.bfloat16), b1,
                   W2.reshape(1, D_), b2)
    return pred[:, 0]
